# baseline (speedup 1.0000x reference)
import functools

import jax
import jax.numpy as jnp
from jax import lax
from jax.experimental import pallas as pl
from jax.experimental.pallas import tpu as pltpu
from jax.experimental.pallas import tpu_sc as plsc

N = 10000
NPAD = 10240
E = 320000
SSUB = 16
MSUB = 32768
ESUB = 131072

_F32 = jnp.float32
_I32 = jnp.int32
_NTILE = 32


def _leaky(x):
    return jax.nn.leaky_relu(x, 0.01)


def _mlp(x, p):
    h = _leaky(x @ p['W1'] + p['b1'])
    h = _leaky(h @ p['W2'] + p['b2'])
    return h @ p['W3'] + p['b3']


def _mp_phase(w, h_hbm, pk_hbm, out_hbm, ebuf, sems, M, F, nprob, rounds, ch, nch):
    def scoped(h_v, agg_v, rdeg_v):
        def start(c, b):
            pltpu.make_async_copy(pk_hbm.at[pl.ds(c * ch, ch)],
                                  ebuf.at[b, pl.ds(0, ch)], sems[b]).start()

        def wait(c, b):
            pltpu.make_async_copy(pk_hbm.at[pl.ds(c * ch, ch)],
                                  ebuf.at[b, pl.ds(0, ch)], sems[b]).wait()

        def edges_sweep(process16):
            start(0, 0)

            def pair(q, _):
                for b in range(2):
                    c = q * 2 + b
                    if b == 0:
                        start(c + 1, 1)
                    else:
                        @pl.when(c + 1 < nch)
                        def _():
                            start(c + 1, 0)
                    wait(c, b)

                    def ebody(i, _):
                        for u in range(4):
                            pk = ebuf[b, pl.ds(i * 64 + u * 16, 16)]
                            s = lax.shift_right_logical(pk, 15)
                            d = lax.bitwise_and(pk, 32767)
                            process16(s, d)
                        return 0

                    lax.fori_loop(0, ch // 64, ebody, 0)
                return 0

            lax.fori_loop(0, nch // 2, pair, 0)

        ones16 = jnp.full((16,), 1.0, _F32)
        zeros16 = jnp.zeros((16,), _F32)
        zidx16 = jnp.zeros((16,), _I32)

        def zdeg(j, _):
            rdeg_v[0, pl.ds(j * 16, 16)] = zeros16
            return 0

        lax.fori_loop(0, M // 16, zdeg, 0)
        edges_sweep(lambda s, d: plsc.addupdate_scatter(rdeg_v, [zidx16, d], ones16))

        def inv(j, _):
            sl = pl.ds(j * 16, 16)
            rdeg_v[0, sl] = 1.0 / jnp.maximum(rdeg_v[0, sl], 1.0)
            return 0

        lax.fori_loop(0, M // 16, inv, 0)

        for f in range(F):
            def zagg(j, _):
                agg_v[f, pl.ds(j * 16, 16)] = zeros16
                return 0

            lax.fori_loop(0, M // 16, zagg, 0)

        def gs(s, d):
            for f in range(F):
                fv = jnp.full((16,), f, _I32)
                g = plsc.load_gather(h_v, [fv, s])
                plsc.addupdate_scatter(agg_v, [fv, d], g)

        def prob_body(p, _):
            rowbase = p * (_NTILE * F) + w * F
            pltpu.sync_copy(h_hbm.at[pl.ds(rowbase, F)], h_v)

            def round_body(r, _):
                edges_sweep(gs)
                for f in range(F):
                    def nb(j, _):
                        sl = pl.ds(j * 16, 16)
                        x = 0.5 * (h_v[f, sl] + agg_v[f, sl] * rdeg_v[0, sl])
                        h_v[f, sl] = jnp.where(x >= 0.0, x, 0.01 * x)
                        agg_v[f, sl] = zeros16
                        return 0

                    lax.fori_loop(0, M // 16, nb, 0)
                return 0

            lax.fori_loop(0, rounds, round_body, 0)
            pltpu.sync_copy(h_v, out_hbm.at[pl.ds(rowbase, F)])
            return 0

        lax.fori_loop(0, nprob, prob_body, 0)

    pl.run_scoped(scoped,
                  pltpu.VMEM((F, M), _F32),
                  pltpu.VMEM((F, M), _F32),
                  pltpu.VMEM((1, M), _F32))


def _build_mp_kernel():
    mesh = plsc.VectorSubcoreMesh(core_axis_name="c", subcore_axis_name="s")
    out_type = (jax.ShapeDtypeStruct((384, NPAD), _F32),
                jax.ShapeDtypeStruct((256, MSUB), _F32))
    scratch = [pltpu.VMEM((2, 8192), _I32),
               pltpu.SemaphoreType.DMA,
               pltpu.SemaphoreType.DMA]

    @functools.partial(pl.kernel, mesh=mesh, out_type=out_type,
                       scratch_types=scratch,
                       compiler_params=pltpu.CompilerParams(
                           needs_layout_passes=False))
    def mp(htop0, pk_main, hsub0, pk_sub, htopf, hsubf, ebuf, sem0, sem1):
        w = lax.axis_index("s") * 2 + lax.axis_index("c")
        sems = (sem0, sem1)
        _mp_phase(w, htop0, pk_main, htopf, ebuf, sems,
                  M=NPAD, F=4, nprob=3, rounds=20, ch=6400, nch=50)
        _mp_phase(w, hsub0, pk_sub, hsubf, ebuf, sems,
                  M=MSUB, F=1, nprob=8, rounds=10, ch=8192, nch=16)

    return mp


_mp_kernel = _build_mp_kernel()


def kernel(node_features, actions, edge_index, angles, sub_graphs, sep_subgraphs, e_offs, gt_edges, post_input, params):
    p = params
    x = node_features
    src = edge_index[0]
    dst = edge_index[1]

    hw1 = _leaky(x @ p['q1']['W1']) @ p['q1']['W2']
    hw2 = _leaky(x @ p['q2']['W1']) @ p['q2']['W2']
    scale = (1.0 + 0.1 * gt_edges)[:, None]
    y1 = _leaky((hw1[src] + hw1[dst]) * scale
                + angles[:, None] * p['q1']['wa'][None, :] + actions[:, None])
    y2 = _leaky((hw2[src] + hw2[dst]) * scale
                + angles[:, None] * p['q2']['wa'][None, :] + actions[:, None])

    eps = 1e-5
    g = p['bn']['gamma']
    b = p['bn']['beta']
    a1 = g / jnp.sqrt(y1.var(0) + eps)
    c1 = b - y1.mean(0) * a1
    a2 = g / jnp.sqrt(y2.var(0) + eps)
    c2 = b - y2.mean(0) * a2

    w13 = a1[:, None] * p['g13']['Win']
    h13 = _leaky(y1 @ w13 + c1 @ p['g13']['Win'])
    w23 = a1[:, None] * p['g23']['Win']
    h23 = _leaky(y1 @ w23 + c1 @ p['g23']['Win'])
    htop0 = jnp.concatenate([h13[:N], h23[:N]], axis=1).T
    htop0 = jnp.pad(htop0, ((0, 0), (0, NPAD - N)))

    y1g = y1[sub_graphs]
    y2g = y2[sub_graphs]
    w12 = a1[:, None] * p['g12']['Win']
    h12 = _leaky(y1g @ w12 + c1 @ p['g12']['Win'])
    w22 = a2[:, None] * p['g22']['Win']
    h22 = _leaky(y2g @ w22 + c2 @ p['g22']['Win'])
    hsub0 = jnp.concatenate([h12, h22], axis=1).T

    pk_main = jnp.bitwise_or(jnp.left_shift(src, 15), dst)
    uv = sep_subgraphs.reshape(-1, 2)
    esrc = jnp.concatenate([uv[:, 0], uv[:, 1]])
    edst = jnp.concatenate([uv[:, 1], uv[:, 0]])
    pk_sub = jnp.bitwise_or(jnp.left_shift(esrc, 15), edst)

    htopf, hsubf = _mp_kernel(htop0, pk_main, hsub0, pk_sub)

    pos = 0.5 ** 20
    neg = 0.005 ** 20
    rest13 = h13[N:] * jnp.where(h13[N:] >= 0, pos, neg)
    rest23 = h23[N:] * jnp.where(h23[N:] >= 0, pos, neg)

    lo = e_offs[:-1, None]
    hi = e_offs[1:, None]
    idx_top = jnp.arange(N)[None, :]
    m_top = ((idx_top >= lo) & (idx_top < hi)).astype(_F32)
    idx_rest = jnp.arange(N, E)[None, :]
    m_rest = ((idx_rest >= lo) & (idx_rest < hi)).astype(_F32)
    cnt = jnp.maximum((e_offs[1:] - e_offs[:-1]).astype(_F32), 1.0)[:, None]

    top_all = htopf.T[:N]
    sums13 = m_top @ top_all[:, :128] + m_rest @ rest13
    sums23 = m_top @ top_all[:, 128:] + m_rest @ rest23
    val1_2 = _mlp((sums13 / cnt) @ p['g13']['Wout'], p['v12'])
    val2_2 = _mlp((sums23 / cnt) @ p['g23']['Wout'], p['v22'])

    sub_all = hsubf.T
    s1 = _leaky(sub_all[:, :128] @ p['g12']['Wout'])
    s2 = _leaky(sub_all[:, 128:] @ p['g22']['Wout'])
    s1 = s1.reshape(-1, SSUB, s1.shape[-1]).mean(1)
    s2 = s2.reshape(-1, SSUB, s2.shape[-1]).mean(1)

    return (jnp.squeeze(_mlp(s1, p['v1'])), jnp.squeeze(val1_2),
            jnp.squeeze(_mlp(s2, p['v2'])), jnp.squeeze(val2_2))


# speedup vs baseline: 1.7402x; 1.7402x over previous
import functools

import jax
import jax.numpy as jnp
from jax import lax
from jax.experimental import pallas as pl
from jax.experimental.pallas import tpu as pltpu
from jax.experimental.pallas import tpu_sc as plsc

N = 10000
NPAD = 10240
E = 320000
SSUB = 16
MSUB = 32768
ESUB = 131072

_F32 = jnp.float32
_I32 = jnp.int32
_NTILE = 32


def _leaky(x):
    return jax.nn.leaky_relu(x, 0.01)


def _mlp(x, p):
    h = _leaky(x @ p['W1'] + p['b1'])
    h = _leaky(h @ p['W2'] + p['b2'])
    return h @ p['W3'] + p['b3']


def _mp_phase(w, h_hbm, pk_hbm, out_hbm, ebuf, sems, M, F, nprob, rounds, ch, nch, grp):
    def scoped(h_v, agg_v, rdeg_v):
        def start(c, b):
            pltpu.make_async_copy(pk_hbm.at[pl.ds(c * ch, ch)],
                                  ebuf.at[b, pl.ds(0, ch)], sems[b]).start()

        def wait(c, b):
            pltpu.make_async_copy(pk_hbm.at[pl.ds(c * ch, ch)],
                                  ebuf.at[b, pl.ds(0, ch)], sems[b]).wait()

        def edges_sweep(process16):
            start(0, 0)

            def pair(q, _):
                for b in range(2):
                    c = q * 2 + b
                    if b == 0:
                        start(c + 1, 1)
                    else:
                        @pl.when(c + 1 < nch)
                        def _():
                            start(c + 1, 0)
                    wait(c, b)

                    ew = grp * 16

                    def ebody(i, _):
                        sds = []
                        for u in range(grp):
                            pk = ebuf[b, pl.ds(i * ew + u * 16, 16)]
                            s = lax.shift_right_logical(pk, 15)
                            d = lax.bitwise_and(pk, 32767)
                            sds.append((s, d))
                        process16(sds)
                        return 0

                    lax.fori_loop(0, ch // ew, ebody, 0)
                return 0

            lax.fori_loop(0, nch // 2, pair, 0)

        ones16 = jnp.full((16,), 1.0, _F32)
        zeros16 = jnp.zeros((16,), _F32)
        zidx16 = jnp.zeros((16,), _I32)

        def zdeg(j, _):
            rdeg_v[0, pl.ds(j * 16, 16)] = zeros16
            return 0

        lax.fori_loop(0, M // 16, zdeg, 0)
        def deg16(sds):
            for s, d in sds:
                plsc.addupdate_scatter(rdeg_v, [zidx16, d], ones16)

        edges_sweep(deg16)

        def inv(j, _):
            sl = pl.ds(j * 16, 16)
            rdeg_v[0, sl] = 1.0 / jnp.maximum(rdeg_v[0, sl], 1.0)
            return 0

        lax.fori_loop(0, M // 16, inv, 0)

        for f in range(F):
            def zagg(j, _):
                agg_v[f, pl.ds(j * 16, 16)] = zeros16
                return 0

            lax.fori_loop(0, M // 16, zagg, 0)

        def gs(sds):
            gathered = []
            for s, d in sds:
                for f in range(F):
                    fv = jnp.full((16,), f, _I32)
                    gathered.append((plsc.load_gather(h_v, [fv, s]), fv, d))
            for g, fv, d in gathered:
                plsc.addupdate_scatter(agg_v, [fv, d], g)

        def prob_body(p, _):
            rowbase = p * (_NTILE * F) + w * F
            pltpu.sync_copy(h_hbm.at[pl.ds(rowbase, F)], h_v)

            def round_body(r, _):
                edges_sweep(gs)
                for f in range(F):
                    def nb(j, _):
                        sl = pl.ds(j * 16, 16)
                        x = 0.5 * (h_v[f, sl] + agg_v[f, sl] * rdeg_v[0, sl])
                        h_v[f, sl] = jnp.where(x >= 0.0, x, 0.01 * x)
                        agg_v[f, sl] = zeros16
                        return 0

                    lax.fori_loop(0, M // 16, nb, 0)
                return 0

            lax.fori_loop(0, rounds, round_body, 0)
            pltpu.sync_copy(h_v, out_hbm.at[pl.ds(rowbase, F)])
            return 0

        lax.fori_loop(0, nprob, prob_body, 0)

    pl.run_scoped(scoped,
                  pltpu.VMEM((F, M), _F32),
                  pltpu.VMEM((F, M), _F32),
                  pltpu.VMEM((1, M), _F32))


def _build_mp_kernel():
    mesh = plsc.VectorSubcoreMesh(core_axis_name="c", subcore_axis_name="s")
    out_type = (jax.ShapeDtypeStruct((384, NPAD), _F32),
                jax.ShapeDtypeStruct((256, MSUB), _F32))
    scratch = [pltpu.VMEM((2, 8192), _I32),
               pltpu.SemaphoreType.DMA,
               pltpu.SemaphoreType.DMA]

    @functools.partial(pl.kernel, mesh=mesh, out_type=out_type,
                       scratch_types=scratch,
                       compiler_params=pltpu.CompilerParams(
                           needs_layout_passes=False))
    def mp(htop0, pk_main, hsub0, pk_sub, htopf, hsubf, ebuf, sem0, sem1):
        w = lax.axis_index("s") * 2 + lax.axis_index("c")
        sems = (sem0, sem1)
        _mp_phase(w, htop0, pk_main, htopf, ebuf, sems,
                  M=NPAD, F=4, nprob=3, rounds=20, ch=6400, nch=50, grp=8)
        _mp_phase(w, hsub0, pk_sub, hsubf, ebuf, sems,
                  M=MSUB, F=1, nprob=8, rounds=10, ch=8192, nch=16, grp=16)

    return mp


_mp_kernel = _build_mp_kernel()


def kernel(node_features, actions, edge_index, angles, sub_graphs, sep_subgraphs, e_offs, gt_edges, post_input, params):
    p = params
    x = node_features
    src = edge_index[0]
    dst = edge_index[1]

    hw1 = _leaky(x @ p['q1']['W1']) @ p['q1']['W2']
    hw2 = _leaky(x @ p['q2']['W1']) @ p['q2']['W2']
    scale = (1.0 + 0.1 * gt_edges)[:, None]
    y1 = _leaky((hw1[src] + hw1[dst]) * scale
                + angles[:, None] * p['q1']['wa'][None, :] + actions[:, None])
    y2 = _leaky((hw2[src] + hw2[dst]) * scale
                + angles[:, None] * p['q2']['wa'][None, :] + actions[:, None])

    eps = 1e-5
    g = p['bn']['gamma']
    b = p['bn']['beta']
    a1 = g / jnp.sqrt(y1.var(0) + eps)
    c1 = b - y1.mean(0) * a1
    a2 = g / jnp.sqrt(y2.var(0) + eps)
    c2 = b - y2.mean(0) * a2

    w13 = a1[:, None] * p['g13']['Win']
    h13 = _leaky(y1 @ w13 + c1 @ p['g13']['Win'])
    w23 = a1[:, None] * p['g23']['Win']
    h23 = _leaky(y1 @ w23 + c1 @ p['g23']['Win'])
    htop0 = jnp.concatenate([h13[:N], h23[:N]], axis=1).T
    htop0 = jnp.pad(htop0, ((0, 0), (0, NPAD - N)))

    y1g = y1[sub_graphs]
    y2g = y2[sub_graphs]
    w12 = a1[:, None] * p['g12']['Win']
    h12 = _leaky(y1g @ w12 + c1 @ p['g12']['Win'])
    w22 = a2[:, None] * p['g22']['Win']
    h22 = _leaky(y2g @ w22 + c2 @ p['g22']['Win'])
    hsub0 = jnp.concatenate([h12, h22], axis=1).T

    pk_main = jnp.bitwise_or(jnp.left_shift(src, 15), dst)
    uv = sep_subgraphs.reshape(-1, 2)
    esrc = jnp.concatenate([uv[:, 0], uv[:, 1]])
    edst = jnp.concatenate([uv[:, 1], uv[:, 0]])
    pk_sub = jnp.bitwise_or(jnp.left_shift(esrc, 15), edst)

    htopf, hsubf = _mp_kernel(htop0, pk_main, hsub0, pk_sub)

    pos = 0.5 ** 20
    neg = 0.005 ** 20
    rest13 = h13[N:] * jnp.where(h13[N:] >= 0, pos, neg)
    rest23 = h23[N:] * jnp.where(h23[N:] >= 0, pos, neg)

    lo = e_offs[:-1, None]
    hi = e_offs[1:, None]
    idx_top = jnp.arange(N)[None, :]
    m_top = ((idx_top >= lo) & (idx_top < hi)).astype(_F32)
    idx_rest = jnp.arange(N, E)[None, :]
    m_rest = ((idx_rest >= lo) & (idx_rest < hi)).astype(_F32)
    cnt = jnp.maximum((e_offs[1:] - e_offs[:-1]).astype(_F32), 1.0)[:, None]

    top_all = htopf.T[:N]
    sums13 = m_top @ top_all[:, :128] + m_rest @ rest13
    sums23 = m_top @ top_all[:, 128:] + m_rest @ rest23
    val1_2 = _mlp((sums13 / cnt) @ p['g13']['Wout'], p['v12'])
    val2_2 = _mlp((sums23 / cnt) @ p['g23']['Wout'], p['v22'])

    sub_all = hsubf.T
    s1 = _leaky(sub_all[:, :128] @ p['g12']['Wout'])
    s2 = _leaky(sub_all[:, 128:] @ p['g22']['Wout'])
    s1 = s1.reshape(-1, SSUB, s1.shape[-1]).mean(1)
    s2 = s2.reshape(-1, SSUB, s2.shape[-1]).mean(1)

    return (jnp.squeeze(_mlp(s1, p['v1'])), jnp.squeeze(val1_2),
            jnp.squeeze(_mlp(s2, p['v2'])), jnp.squeeze(val2_2))


# speedup vs baseline: 1.7481x; 1.0045x over previous
import functools

import jax
import jax.numpy as jnp
from jax import lax
from jax.experimental import pallas as pl
from jax.experimental.pallas import tpu as pltpu
from jax.experimental.pallas import tpu_sc as plsc

N = 10000
NPAD = 10240
E = 320000
SSUB = 16
MSUB = 32768
ESUB = 131072

_F32 = jnp.float32
_I32 = jnp.int32
_NTILE = 32


def _leaky(x):
    return jax.nn.leaky_relu(x, 0.01)


def _mlp(x, p):
    h = _leaky(x @ p['W1'] + p['b1'])
    h = _leaky(h @ p['W2'] + p['b2'])
    return h @ p['W3'] + p['b3']


def _mp_phase(w, h_hbm, pk_hbm, out_hbm, ebuf, sems, M, F, nprob, rounds, ch, nch, grp):
    def scoped(h_v, agg_v, rdeg_v):
        def start(c, b):
            pltpu.make_async_copy(pk_hbm.at[pl.ds(c * ch, ch)],
                                  ebuf.at[b, pl.ds(0, ch)], sems[b]).start()

        def wait(c, b):
            pltpu.make_async_copy(pk_hbm.at[pl.ds(c * ch, ch)],
                                  ebuf.at[b, pl.ds(0, ch)], sems[b]).wait()

        def edges_sweep(gather16, scatter16):
            start(0, 0)
            ew = grp * 16

            def pair(q, _):
                for b in range(2):
                    c = q * 2 + b
                    if b == 0:
                        start(c + 1, 1)
                    else:
                        @pl.when(c + 1 < nch)
                        def _():
                            start(c + 1, 0)
                    wait(c, b)

                    def gbatch(i):
                        acc = []
                        for u in range(grp):
                            pk = ebuf[b, pl.ds(i * ew + u * 16, 16)]
                            s = lax.shift_right_logical(pk, 15)
                            d = lax.bitwise_and(pk, 32767)
                            acc.extend(gather16(s, d))
                        return tuple(acc)

                    def ebody(i, prev):
                        cur = gbatch(i)
                        scatter16(prev)
                        return cur

                    last = lax.fori_loop(1, ch // ew, ebody, gbatch(0))
                    scatter16(last)
                return 0

            lax.fori_loop(0, nch // 2, pair, 0)

        ones16 = jnp.full((16,), 1.0, _F32)
        zeros16 = jnp.zeros((16,), _F32)
        zidx16 = jnp.zeros((16,), _I32)

        def zdeg(j, _):
            rdeg_v[0, pl.ds(j * 16, 16)] = zeros16
            return 0

        lax.fori_loop(0, M // 16, zdeg, 0)
        def deg_g(s, d):
            return [d]

        def deg_s(batch):
            for d in batch:
                plsc.addupdate_scatter(rdeg_v, [zidx16, d], ones16)

        edges_sweep(deg_g, deg_s)

        def inv(j, _):
            sl = pl.ds(j * 16, 16)
            rdeg_v[0, sl] = 1.0 / jnp.maximum(rdeg_v[0, sl], 1.0)
            return 0

        lax.fori_loop(0, M // 16, inv, 0)

        for f in range(F):
            def zagg(j, _):
                agg_v[f, pl.ds(j * 16, 16)] = zeros16
                return 0

            lax.fori_loop(0, M // 16, zagg, 0)

        fvs = [jnp.full((16,), f, _I32) for f in range(F)]

        def gs_g(s, d):
            out = [plsc.load_gather(h_v, [fvs[f], s]) for f in range(F)]
            out.append(d)
            return out

        def gs_s(batch):
            for u in range(grp):
                d = batch[u * (F + 1) + F]
                for f in range(F):
                    plsc.addupdate_scatter(agg_v, [fvs[f], d],
                                           batch[u * (F + 1) + f])

        def prob_body(p, _):
            rowbase = p * (_NTILE * F) + w * F
            pltpu.sync_copy(h_hbm.at[pl.ds(rowbase, F)], h_v)

            def round_body(r, _):
                edges_sweep(gs_g, gs_s)
                for f in range(F):
                    def nb(j, _):
                        sl = pl.ds(j * 16, 16)
                        x = 0.5 * (h_v[f, sl] + agg_v[f, sl] * rdeg_v[0, sl])
                        h_v[f, sl] = jnp.where(x >= 0.0, x, 0.01 * x)
                        agg_v[f, sl] = zeros16
                        return 0

                    lax.fori_loop(0, M // 16, nb, 0)
                return 0

            lax.fori_loop(0, rounds, round_body, 0)
            pltpu.sync_copy(h_v, out_hbm.at[pl.ds(rowbase, F)])
            return 0

        lax.fori_loop(0, nprob, prob_body, 0)

    pl.run_scoped(scoped,
                  pltpu.VMEM((F, M), _F32),
                  pltpu.VMEM((F, M), _F32),
                  pltpu.VMEM((1, M), _F32))


def _build_mp_kernel():
    mesh = plsc.VectorSubcoreMesh(core_axis_name="c", subcore_axis_name="s")
    out_type = (jax.ShapeDtypeStruct((384, NPAD), _F32),
                jax.ShapeDtypeStruct((256, MSUB), _F32))
    scratch = [pltpu.VMEM((2, 8192), _I32),
               pltpu.SemaphoreType.DMA,
               pltpu.SemaphoreType.DMA]

    @functools.partial(pl.kernel, mesh=mesh, out_type=out_type,
                       scratch_types=scratch,
                       compiler_params=pltpu.CompilerParams(
                           needs_layout_passes=False))
    def mp(htop0, pk_main, hsub0, pk_sub, htopf, hsubf, ebuf, sem0, sem1):
        w = lax.axis_index("s") * 2 + lax.axis_index("c")
        sems = (sem0, sem1)
        _mp_phase(w, htop0, pk_main, htopf, ebuf, sems,
                  M=NPAD, F=4, nprob=3, rounds=20, ch=6400, nch=50, grp=4)
        _mp_phase(w, hsub0, pk_sub, hsubf, ebuf, sems,
                  M=MSUB, F=1, nprob=8, rounds=10, ch=8192, nch=16, grp=8)

    return mp


_mp_kernel = _build_mp_kernel()


def kernel(node_features, actions, edge_index, angles, sub_graphs, sep_subgraphs, e_offs, gt_edges, post_input, params):
    p = params
    x = node_features
    src = edge_index[0]
    dst = edge_index[1]

    hw1 = _leaky(x @ p['q1']['W1']) @ p['q1']['W2']
    hw2 = _leaky(x @ p['q2']['W1']) @ p['q2']['W2']
    scale = (1.0 + 0.1 * gt_edges)[:, None]
    y1 = _leaky((hw1[src] + hw1[dst]) * scale
                + angles[:, None] * p['q1']['wa'][None, :] + actions[:, None])
    y2 = _leaky((hw2[src] + hw2[dst]) * scale
                + angles[:, None] * p['q2']['wa'][None, :] + actions[:, None])

    eps = 1e-5
    g = p['bn']['gamma']
    b = p['bn']['beta']
    a1 = g / jnp.sqrt(y1.var(0) + eps)
    c1 = b - y1.mean(0) * a1
    a2 = g / jnp.sqrt(y2.var(0) + eps)
    c2 = b - y2.mean(0) * a2

    w13 = a1[:, None] * p['g13']['Win']
    h13 = _leaky(y1 @ w13 + c1 @ p['g13']['Win'])
    w23 = a1[:, None] * p['g23']['Win']
    h23 = _leaky(y1 @ w23 + c1 @ p['g23']['Win'])
    htop0 = jnp.concatenate([h13[:N], h23[:N]], axis=1).T
    htop0 = jnp.pad(htop0, ((0, 0), (0, NPAD - N)))

    y1g = y1[sub_graphs]
    y2g = y2[sub_graphs]
    w12 = a1[:, None] * p['g12']['Win']
    h12 = _leaky(y1g @ w12 + c1 @ p['g12']['Win'])
    w22 = a2[:, None] * p['g22']['Win']
    h22 = _leaky(y2g @ w22 + c2 @ p['g22']['Win'])
    hsub0 = jnp.concatenate([h12, h22], axis=1).T

    pk_main = jnp.bitwise_or(jnp.left_shift(src, 15), dst)
    uv = sep_subgraphs.reshape(-1, 2)
    esrc = jnp.concatenate([uv[:, 0], uv[:, 1]])
    edst = jnp.concatenate([uv[:, 1], uv[:, 0]])
    pk_sub = jnp.bitwise_or(jnp.left_shift(esrc, 15), edst)

    htopf, hsubf = _mp_kernel(htop0, pk_main, hsub0, pk_sub)

    pos = 0.5 ** 20
    neg = 0.005 ** 20
    rest13 = h13[N:] * jnp.where(h13[N:] >= 0, pos, neg)
    rest23 = h23[N:] * jnp.where(h23[N:] >= 0, pos, neg)

    lo = e_offs[:-1, None]
    hi = e_offs[1:, None]
    idx_top = jnp.arange(N)[None, :]
    m_top = ((idx_top >= lo) & (idx_top < hi)).astype(_F32)
    idx_rest = jnp.arange(N, E)[None, :]
    m_rest = ((idx_rest >= lo) & (idx_rest < hi)).astype(_F32)
    cnt = jnp.maximum((e_offs[1:] - e_offs[:-1]).astype(_F32), 1.0)[:, None]

    top_all = htopf.T[:N]
    sums13 = m_top @ top_all[:, :128] + m_rest @ rest13
    sums23 = m_top @ top_all[:, 128:] + m_rest @ rest23
    val1_2 = _mlp((sums13 / cnt) @ p['g13']['Wout'], p['v12'])
    val2_2 = _mlp((sums23 / cnt) @ p['g23']['Wout'], p['v22'])

    sub_all = hsubf.T
    s1 = _leaky(sub_all[:, :128] @ p['g12']['Wout'])
    s2 = _leaky(sub_all[:, 128:] @ p['g22']['Wout'])
    s1 = s1.reshape(-1, SSUB, s1.shape[-1]).mean(1)
    s2 = s2.reshape(-1, SSUB, s2.shape[-1]).mean(1)

    return (jnp.squeeze(_mlp(s1, p['v1'])), jnp.squeeze(val1_2),
            jnp.squeeze(_mlp(s2, p['v2'])), jnp.squeeze(val2_2))


# speedup vs baseline: 1.7638x; 1.0090x over previous
import functools

import jax
import jax.numpy as jnp
from jax import lax
from jax.experimental import pallas as pl
from jax.experimental.pallas import tpu as pltpu
from jax.experimental.pallas import tpu_sc as plsc

N = 10000
NPAD = 10240
E = 320000
SSUB = 16
MSUB = 32768
ESUB = 131072

_F32 = jnp.float32
_I32 = jnp.int32
_NTILE = 32


def _leaky(x):
    return jax.nn.leaky_relu(x, 0.01)


def _mlp(x, p):
    h = _leaky(x @ p['W1'] + p['b1'])
    h = _leaky(h @ p['W2'] + p['b2'])
    return h @ p['W3'] + p['b3']


def _mp_phase(w, h_hbm, pk_hbm, out_hbm, ebuf, sems, M, F, nprob, rounds, ch, nch, grp):
    def scoped(h_v, agg_v, rdeg_v):
        def start(c, b):
            pltpu.make_async_copy(pk_hbm.at[pl.ds(c * ch, ch)],
                                  ebuf.at[b, pl.ds(0, ch)], sems[b]).start()

        def wait(c, b):
            pltpu.make_async_copy(pk_hbm.at[pl.ds(c * ch, ch)],
                                  ebuf.at[b, pl.ds(0, ch)], sems[b]).wait()

        def edges_sweep(gather16, scatter16):
            start(0, 0)
            ew = grp * 16

            def pair(q, _):
                for b in range(2):
                    c = q * 2 + b
                    if b == 0:
                        start(c + 1, 1)
                    else:
                        @pl.when(c + 1 < nch)
                        def _():
                            start(c + 1, 0)
                    wait(c, b)

                    def gbatch(i):
                        acc = []
                        for u in range(grp):
                            pk = ebuf[b, pl.ds(i * ew + u * 16, 16)]
                            s = lax.shift_right_logical(pk, 15)
                            d = lax.bitwise_and(pk, 32767)
                            acc.extend(gather16(s, d))
                        return tuple(acc)

                    def ebody(i, prev):
                        cur = gbatch(i)
                        scatter16(prev)
                        return cur

                    last = lax.fori_loop(1, ch // ew, ebody, gbatch(0))
                    scatter16(last)
                return 0

            lax.fori_loop(0, nch // 2, pair, 0)

        ones16 = jnp.full((16,), 1.0, _F32)
        zeros16 = jnp.zeros((16,), _F32)
        zidx16 = jnp.zeros((16,), _I32)

        def zdeg(j, _):
            rdeg_v[0, pl.ds(j * 16, 16)] = zeros16
            return 0

        lax.fori_loop(0, M // 16, zdeg, 0)
        def deg_g(s, d):
            return [d]

        def deg_s(batch):
            for d in batch:
                plsc.addupdate_scatter(rdeg_v, [zidx16, d], ones16)

        edges_sweep(deg_g, deg_s)

        def inv(j, _):
            sl = pl.ds(j * 16, 16)
            rdeg_v[0, sl] = 1.0 / jnp.maximum(rdeg_v[0, sl], 1.0)
            return 0

        lax.fori_loop(0, M // 16, inv, 0)

        for f in range(F):
            def zagg(j, _):
                agg_v[f, pl.ds(j * 16, 16)] = zeros16
                return 0

            lax.fori_loop(0, M // 16, zagg, 0)

        fvs = [jnp.full((16,), f, _I32) for f in range(F)]

        def gs_g(s, d):
            out = [plsc.load_gather(h_v, [fvs[f], s]) for f in range(F)]
            out.append(d)
            return out

        def gs_s(batch):
            for u in range(grp):
                d = batch[u * (F + 1) + F]
                for f in range(F):
                    plsc.addupdate_scatter(agg_v, [fvs[f], d],
                                           batch[u * (F + 1) + f])

        def prob_body(p, _):
            rowbase = p * (_NTILE * F) + w * F
            pltpu.sync_copy(h_hbm.at[pl.ds(rowbase, F)], h_v)

            def round_body(r, _):
                edges_sweep(gs_g, gs_s)
                for f in range(F):
                    def nb(j, _):
                        sl = pl.ds(j * 16, 16)
                        x = 0.5 * (h_v[f, sl] + agg_v[f, sl] * rdeg_v[0, sl])
                        h_v[f, sl] = jnp.where(x >= 0.0, x, 0.01 * x)
                        agg_v[f, sl] = zeros16
                        return 0

                    lax.fori_loop(0, M // 16, nb, 0)
                return 0

            lax.fori_loop(0, rounds, round_body, 0)
            pltpu.sync_copy(h_v, out_hbm.at[pl.ds(rowbase, F)])
            return 0

        lax.fori_loop(0, nprob, prob_body, 0)

    pl.run_scoped(scoped,
                  pltpu.VMEM((F, M), _F32),
                  pltpu.VMEM((F, M), _F32),
                  pltpu.VMEM((1, M), _F32))


def _build_mp_kernel():
    mesh = plsc.VectorSubcoreMesh(core_axis_name="c", subcore_axis_name="s")
    out_type = (jax.ShapeDtypeStruct((384, NPAD), _F32),
                jax.ShapeDtypeStruct((256, MSUB), _F32))
    scratch = [pltpu.VMEM((2, 8192), _I32),
               pltpu.SemaphoreType.DMA,
               pltpu.SemaphoreType.DMA]

    @functools.partial(pl.kernel, mesh=mesh, out_type=out_type,
                       scratch_types=scratch,
                       compiler_params=pltpu.CompilerParams(
                           needs_layout_passes=False))
    def mp(htop0, pk_main, hsub0, pk_sub, htopf, hsubf, ebuf, sem0, sem1):
        w = lax.axis_index("s") * 2 + lax.axis_index("c")
        sems = (sem0, sem1)
        _mp_phase(w, htop0, pk_main, htopf, ebuf, sems,
                  M=NPAD, F=4, nprob=3, rounds=20, ch=6400, nch=50, grp=5)
        _mp_phase(w, hsub0, pk_sub, hsubf, ebuf, sems,
                  M=MSUB, F=1, nprob=8, rounds=10, ch=8192, nch=16, grp=16)

    return mp


_mp_kernel = _build_mp_kernel()


def kernel(node_features, actions, edge_index, angles, sub_graphs, sep_subgraphs, e_offs, gt_edges, post_input, params):
    p = params
    x = node_features
    src = edge_index[0]
    dst = edge_index[1]

    hw1 = _leaky(x @ p['q1']['W1']) @ p['q1']['W2']
    hw2 = _leaky(x @ p['q2']['W1']) @ p['q2']['W2']
    scale = (1.0 + 0.1 * gt_edges)[:, None]
    y1 = _leaky((hw1[src] + hw1[dst]) * scale
                + angles[:, None] * p['q1']['wa'][None, :] + actions[:, None])
    y2 = _leaky((hw2[src] + hw2[dst]) * scale
                + angles[:, None] * p['q2']['wa'][None, :] + actions[:, None])

    eps = 1e-5
    g = p['bn']['gamma']
    b = p['bn']['beta']
    a1 = g / jnp.sqrt(y1.var(0) + eps)
    c1 = b - y1.mean(0) * a1
    a2 = g / jnp.sqrt(y2.var(0) + eps)
    c2 = b - y2.mean(0) * a2

    w13 = a1[:, None] * p['g13']['Win']
    h13 = _leaky(y1 @ w13 + c1 @ p['g13']['Win'])
    w23 = a1[:, None] * p['g23']['Win']
    h23 = _leaky(y1 @ w23 + c1 @ p['g23']['Win'])
    htop0 = jnp.concatenate([h13[:N], h23[:N]], axis=1).T
    htop0 = jnp.pad(htop0, ((0, 0), (0, NPAD - N)))

    y1g = y1[sub_graphs]
    y2g = y2[sub_graphs]
    w12 = a1[:, None] * p['g12']['Win']
    h12 = _leaky(y1g @ w12 + c1 @ p['g12']['Win'])
    w22 = a2[:, None] * p['g22']['Win']
    h22 = _leaky(y2g @ w22 + c2 @ p['g22']['Win'])
    hsub0 = jnp.concatenate([h12, h22], axis=1).T

    pk_main = jnp.bitwise_or(jnp.left_shift(src, 15), dst)
    uv = sep_subgraphs.reshape(-1, 2)
    esrc = jnp.concatenate([uv[:, 0], uv[:, 1]])
    edst = jnp.concatenate([uv[:, 1], uv[:, 0]])
    pk_sub = jnp.bitwise_or(jnp.left_shift(esrc, 15), edst)

    htopf, hsubf = _mp_kernel(htop0, pk_main, hsub0, pk_sub)

    pos = 0.5 ** 20
    neg = 0.005 ** 20
    rest13 = h13[N:] * jnp.where(h13[N:] >= 0, pos, neg)
    rest23 = h23[N:] * jnp.where(h23[N:] >= 0, pos, neg)

    lo = e_offs[:-1, None]
    hi = e_offs[1:, None]
    idx_top = jnp.arange(N)[None, :]
    m_top = ((idx_top >= lo) & (idx_top < hi)).astype(_F32)
    idx_rest = jnp.arange(N, E)[None, :]
    m_rest = ((idx_rest >= lo) & (idx_rest < hi)).astype(_F32)
    cnt = jnp.maximum((e_offs[1:] - e_offs[:-1]).astype(_F32), 1.0)[:, None]

    top_all = htopf.T[:N]
    sums13 = m_top @ top_all[:, :128] + m_rest @ rest13
    sums23 = m_top @ top_all[:, 128:] + m_rest @ rest23
    val1_2 = _mlp((sums13 / cnt) @ p['g13']['Wout'], p['v12'])
    val2_2 = _mlp((sums23 / cnt) @ p['g23']['Wout'], p['v22'])

    sub_all = hsubf.T
    s1 = _leaky(sub_all[:, :128] @ p['g12']['Wout'])
    s2 = _leaky(sub_all[:, 128:] @ p['g22']['Wout'])
    s1 = s1.reshape(-1, SSUB, s1.shape[-1]).mean(1)
    s2 = s2.reshape(-1, SSUB, s2.shape[-1]).mean(1)

    return (jnp.squeeze(_mlp(s1, p['v1'])), jnp.squeeze(val1_2),
            jnp.squeeze(_mlp(s2, p['v2'])), jnp.squeeze(val2_2))
